# baseline (speedup 1.0000x reference)
import functools

import jax
import jax.numpy as jnp
from jax import lax
from jax.experimental import pallas as pl
from jax.experimental.pallas import tpu as pltpu
from jax.experimental.pallas import tpu_sc as plsc

_INFO = plsc.get_sparse_core_info()
_NC = _INFO.num_cores
_NS = _INFO.num_subcores
_NW = _NC * _NS

_CHUNK = 128
_NBUF = 4
_L = 16


def _gather_grid(dim, chunks_per_w, bblocks, total_nodes):
    mesh = plsc.VectorSubcoreMesh(core_axis_name="c", subcore_axis_name="s")
    dsub = dim // 8
    hist = chunks_per_w * _NW // bblocks

    @functools.partial(
        pl.kernel,
        mesh=mesh,
        compiler_params=pltpu.CompilerParams(use_tc_tiling_on_sc=False,
                                             needs_layout_passes=False),
        out_type=jax.ShapeDtypeStruct((hist, dsub, bblocks, 8, _CHUNK),
                                      jnp.float32),
        scratch_types=[
            pltpu.VMEM((chunks_per_w, _CHUNK), jnp.int32),
            pltpu.VMEM((_NBUF, _CHUNK, dim), jnp.float32),
            pltpu.VMEM((_NBUF, dsub, 8, _CHUNK), jnp.float32),
            pltpu.SemaphoreType.DMA,
            pltpu.SemaphoreType.DMA,
        ],
    )
    def body(ids_hbm, table_hbm, out_hbm, idx_v, rows_v, perm_v, gsem, ssem):
        wid = lax.axis_index("s") * _NC + lax.axis_index("c")
        row0 = wid * chunks_per_w

        def gather(j, b):
            pltpu.async_copy(table_hbm.at[idx_v.at[j]], rows_v.at[b], gsem)

        def gather_wait(b):
            pltpu.make_async_copy(table_hbm.at[idx_v.at[0]], rows_v.at[b],
                                  gsem).wait()

        def store(j, b):
            h = (row0 + j) // bblocks
            tc = (row0 + j) % bblocks
            pltpu.async_copy(perm_v.at[b], out_hbm.at[h, :, tc], ssem)

        def store_wait(b):
            pltpu.make_async_copy(perm_v.at[b], out_hbm.at[0, :, 0],
                                  ssem).wait()

        def transpose(b):
            lanes = lax.iota(jnp.int32, _L)
            bsplat = jnp.full((_L,), b, jnp.int32)

            @pl.loop(0, dim)
            def _(d):
                dspl = jnp.full((_L,), d, jnp.int32)
                tr = d // 8
                s = d % 8
                for l0 in range(0, _CHUNK, _L):
                    val = plsc.load_gather(rows_v, [bsplat, lanes + l0, dspl])
                    perm_v[b, tr, s, pl.ds(l0, _L)] = val

        pltpu.sync_copy(ids_hbm.at[pl.ds(row0, chunks_per_w)], idx_v)
        for b in range(_NBUF):
            gather(b, b)

        ngroups = chunks_per_w // _NBUF

        @pl.loop(0, ngroups - 1)
        def _(t):
            g = t * _NBUF
            for b in range(_NBUF):
                gather_wait(b)

                @pl.when(t > 0)
                def _():
                    store_wait(b)

                transpose(b)
                store(g + b, b)
                gather(g + _NBUF + b, b)

        gl = (ngroups - 1) * _NBUF
        for b in range(_NBUF):
            gather_wait(b)
            store_wait(b)
            transpose(b)
            store(gl + b, b)
        for b in range(_NBUF):
            store_wait(b)

    return body


def kernel(node_ids, embeddings):
    batch, hist = node_ids.shape
    nodes, dim = embeddings.shape
    total = batch * hist
    per_w = total // _NW
    chunks_per_w = per_w // _CHUNK
    bblocks = batch // _CHUNK

    ids2d = node_ids.T.reshape(total // _CHUNK, _CHUNK)
    tab128 = lax.optimization_barrier(embeddings.reshape(-1, 128))
    tab_lin = tab128.reshape(nodes, dim)

    out5 = _gather_grid(dim, chunks_per_w, bblocks, nodes)(ids2d, tab_lin)
    return out5.transpose(2, 4, 0, 1, 3).reshape(batch, hist, dim)


# speedup vs baseline: 2.0395x; 2.0395x over previous
import functools

import jax
import jax.numpy as jnp
from jax import lax
from jax.experimental import pallas as pl
from jax.experimental.pallas import tpu as pltpu
from jax.experimental.pallas import tpu_sc as plsc

_INFO = plsc.get_sparse_core_info()
_NC = _INFO.num_cores
_NS = _INFO.num_subcores
_NW = _NC * _NS

_CHUNK = 128
_NBUF = 8


def _gather_grid(total, dim, chunks_per_w):
    mesh = plsc.VectorSubcoreMesh(core_axis_name="c", subcore_axis_name="s")

    @functools.partial(
        pl.kernel,
        mesh=mesh,
        compiler_params=pltpu.CompilerParams(use_tc_tiling_on_sc=False),
        out_type=jax.ShapeDtypeStruct((total // _CHUNK, _CHUNK, 2 * dim),
                                      jnp.float32),
        scratch_types=[
            pltpu.VMEM((chunks_per_w, _CHUNK), jnp.int32),
            pltpu.VMEM((_NBUF, _CHUNK, dim), jnp.float32),
            pltpu.SemaphoreType.DMA,
            pltpu.SemaphoreType.DMA,
        ],
    )
    def body(ids_hbm, table_hbm, out_hbm, idx_v, rows_v, gsem, ssem):
        wid = lax.axis_index("s") * _NC + lax.axis_index("c")
        row0 = wid * chunks_per_w

        def gather(j, b):
            pltpu.async_copy(table_hbm.at[idx_v.at[j]], rows_v.at[b], gsem)

        def gather_wait(b):
            pltpu.make_async_copy(table_hbm.at[idx_v.at[0]], rows_v.at[b],
                                  gsem).wait()

        def store(j, b):
            pltpu.async_copy(rows_v.at[b],
                             out_hbm.at[row0 + j, :, pl.ds(0, dim)], ssem)

        def store_wait(b):
            pltpu.make_async_copy(rows_v.at[b], out_hbm.at[0, :, pl.ds(0, dim)],
                                  ssem).wait()

        pltpu.sync_copy(ids_hbm.at[pl.ds(row0, chunks_per_w)], idx_v)
        for b in range(_NBUF):
            gather(b, b)

        @pl.loop(0, (chunks_per_w - _NBUF) // _NBUF)
        def _(t):
            g = t * _NBUF
            for b in range(_NBUF):
                gather_wait(b)
                store(g + b, b)
            for b in range(_NBUF):
                store_wait(b)
                gather(g + _NBUF + b, b)

        gl = chunks_per_w - _NBUF
        for b in range(_NBUF):
            gather_wait(b)
            store(gl + b, b)
        for b in range(_NBUF):
            store_wait(b)

    return body


def kernel(node_ids, embeddings):
    batch, hist = node_ids.shape
    nodes, dim = embeddings.shape
    total = batch * hist
    per_w = total // _NW
    chunks_per_w = per_w // _CHUNK

    ids2d = node_ids.T.reshape(total // _CHUNK, _CHUNK)
    tab128 = lax.optimization_barrier(embeddings.reshape(-1, 2 * dim))
    tab_lin = tab128.reshape(nodes, dim)

    out3 = _gather_grid(total, dim, chunks_per_w)(ids2d, tab_lin)
    out_pad = out3.reshape(total, 2 * dim)
    return (out_pad[:, :dim].reshape(hist, batch, dim).transpose(1, 0, 2))
